# Initial kernel scaffold; baseline (speedup 1.0000x reference)
import jax
import jax.numpy as jnp
from jax.experimental import pallas as pl


def kernel(hidden_states, score_w, score_b, amp_w, amp_b):
    raise NotImplementedError("write your pallas kernel here")


# speedup vs baseline: 22.4369x; 22.4369x over previous
import functools

import jax
import jax.numpy as jnp
from jax.experimental import pallas as pl
from jax.experimental.pallas import tpu as pltpu

INPUT_DIM = 768
NUM_PRIMES = 8192
K = 8
AMP_DIM = 4
AK = K * AMP_DIM

ROW_BLOCK = 256
PRIME_TILE = 2048
NUM_TILES = NUM_PRIMES // PRIME_TILE
SLOT = 128


def _fused(hs_ref, sw_ref, sb_ref, aw_ref, ab_ref,
           idx_ref, amp_ref, cv_ref, ci_ref):
    j = pl.program_id(1)
    hs = hs_ref[...]
    scores = jax.lax.dot_general(
        hs, sw_ref[...], (((1,), (1,)), ((), ())),
        preferred_element_type=jnp.float32)
    scores = scores + sb_ref[...]

    lane = jax.lax.broadcasted_iota(jnp.int32, (ROW_BLOCK, PRIME_TILE), 1)
    base = j * PRIME_TILE
    vals, idxs = [], []
    x = scores
    for _ in range(K):
        m = jnp.max(x, axis=1, keepdims=True)
        hit = x == m
        pos = jnp.min(jnp.where(hit, lane, PRIME_TILE), axis=1, keepdims=True)
        vals.append(m)
        idxs.append(pos + base)
        x = jnp.where(lane == pos, -jnp.inf, x)
    vpad = jnp.full((ROW_BLOCK, SLOT - K), -jnp.inf, dtype=jnp.float32)
    ipad = jnp.zeros((ROW_BLOCK, SLOT - K), dtype=jnp.int32)
    cv_ref[:, pl.ds(j * SLOT, SLOT)] = jnp.concatenate(vals + [vpad], axis=1)
    ci_ref[:, pl.ds(j * SLOT, SLOT)] = jnp.concatenate(idxs + [ipad], axis=1)

    @pl.when(j == NUM_TILES - 1)
    def _merge():
        nc = NUM_TILES * SLOT
        cv = cv_ref[...]
        ci = ci_ref[...]
        slot = jax.lax.broadcasted_iota(jnp.int32, (ROW_BLOCK, nc), 1)
        x2 = cv
        fv, fi = [], []
        for _ in range(K):
            m = jnp.max(x2, axis=1, keepdims=True)
            pos = jnp.min(jnp.where(x2 == m, slot, nc), axis=1, keepdims=True)
            sel = slot == pos
            fv.append(m)
            fi.append(jnp.sum(jnp.where(sel, ci, 0), axis=1, keepdims=True))
            x2 = jnp.where(sel, -jnp.inf, x2)
        topv = jnp.concatenate(fv, axis=1)
        idx_ref[...] = jnp.concatenate(fi, axis=1)

        w = jnp.exp(topv - topv[:, :1])
        w = w / jnp.sum(w, axis=1, keepdims=True)

        amps = jax.lax.dot_general(
            hs, aw_ref[...], (((1,), (1,)), ((), ())),
            preferred_element_type=jnp.float32) + ab_ref[...]

        r8 = jax.lax.broadcasted_iota(jnp.int32, (K, AK), 0)
        c32 = jax.lax.broadcasted_iota(jnp.int32, (K, AK), 1)
        expand = (c32 // AMP_DIM == r8).astype(jnp.float32)
        w32 = jax.lax.dot_general(
            w, expand, (((1,), (0,)), ((), ())),
            preferred_element_type=jnp.float32)
        wa = amps * w32
        g = wa * wa
        p = jax.lax.broadcasted_iota(jnp.int32, (AK, AK), 0)
        q = jax.lax.broadcasted_iota(jnp.int32, (AK, AK), 1)
        gsum = (p // AMP_DIM == q // AMP_DIM).astype(jnp.float32)
        n2 = jax.lax.dot_general(
            g, gsum, (((1,), (0,)), ((), ())),
            preferred_element_type=jnp.float32)
        amp_ref[...] = wa / jnp.maximum(jnp.sqrt(n2), 1e-12)


@functools.partial(jax.jit, static_argnames=())
def kernel(hidden_states, score_w, score_b, amp_w, amp_b):
    batch, seq, d = hidden_states.shape
    rows = batch * seq
    hs2 = hidden_states.reshape(rows, d)
    sb2 = score_b.reshape(1, NUM_PRIMES)
    ab2 = amp_b.reshape(1, AK)
    nr = rows // ROW_BLOCK

    idx_out, amp_out = pl.pallas_call(
        _fused,
        grid=(nr, NUM_TILES),
        in_specs=[
            pl.BlockSpec((ROW_BLOCK, d), lambda i, j: (i, 0)),
            pl.BlockSpec((PRIME_TILE, d), lambda i, j: (j, 0)),
            pl.BlockSpec((1, PRIME_TILE), lambda i, j: (0, j)),
            pl.BlockSpec((AK, d), lambda i, j: (0, 0)),
            pl.BlockSpec((1, AK), lambda i, j: (0, 0)),
        ],
        out_specs=[
            pl.BlockSpec((ROW_BLOCK, K), lambda i, j: (i, 0)),
            pl.BlockSpec((ROW_BLOCK, AK), lambda i, j: (i, 0)),
        ],
        out_shape=[
            jax.ShapeDtypeStruct((rows, K), jnp.int32),
            jax.ShapeDtypeStruct((rows, AK), jnp.float32),
        ],
        scratch_shapes=[
            pltpu.VMEM((ROW_BLOCK, NUM_TILES * SLOT), jnp.float32),
            pltpu.VMEM((ROW_BLOCK, NUM_TILES * SLOT), jnp.int32),
        ],
        compiler_params=pltpu.CompilerParams(
            dimension_semantics=("parallel", "arbitrary")),
    )(hs2, score_w, sb2, amp_w, ab2)

    topk_indices = idx_out.reshape(batch, seq, K)
    amps = amp_out.reshape(batch, seq, K, AMP_DIM)
    return (topk_indices, amps)


# speedup vs baseline: 26.3379x; 1.1739x over previous
import functools

import jax
import jax.numpy as jnp
from jax.experimental import pallas as pl
from jax.experimental.pallas import tpu as pltpu

INPUT_DIM = 768
NUM_PRIMES = 8192
K = 8
AMP_DIM = 4
AK = K * AMP_DIM

ROW_BLOCK = 1024
PRIME_TILE = 2048
NUM_TILES = NUM_PRIMES // PRIME_TILE
SLOT = 128


def _fused(hs_ref, sw_ref, sb_ref, aw_ref, ab_ref,
           idx_ref, amp_ref, cv_ref, ci_ref):
    j = pl.program_id(1)
    hs = hs_ref[...]
    scores = jax.lax.dot_general(
        hs, sw_ref[...], (((1,), (1,)), ((), ())),
        preferred_element_type=jnp.float32)
    scores = scores + sb_ref[...]

    lane = jax.lax.broadcasted_iota(jnp.int32, (ROW_BLOCK, PRIME_TILE), 1)
    base = j * PRIME_TILE
    vals, idxs = [], []
    x = scores
    for _ in range(K):
        m = jnp.max(x, axis=1, keepdims=True)
        hit = x == m
        pos = jnp.min(jnp.where(hit, lane, PRIME_TILE), axis=1, keepdims=True)
        vals.append(m)
        idxs.append(pos + base)
        x = jnp.where(lane == pos, -jnp.inf, x)
    vpad = jnp.full((ROW_BLOCK, SLOT - K), -jnp.inf, dtype=jnp.float32)
    ipad = jnp.zeros((ROW_BLOCK, SLOT - K), dtype=jnp.int32)
    cv_ref[:, pl.ds(j * SLOT, SLOT)] = jnp.concatenate(vals + [vpad], axis=1)
    ci_ref[:, pl.ds(j * SLOT, SLOT)] = jnp.concatenate(idxs + [ipad], axis=1)

    @pl.when(j == NUM_TILES - 1)
    def _merge():
        nc = NUM_TILES * SLOT
        cv = cv_ref[...]
        ci = ci_ref[...]
        slot = jax.lax.broadcasted_iota(jnp.int32, (ROW_BLOCK, nc), 1)
        x2 = cv
        fv, fi = [], []
        for _ in range(K):
            m = jnp.max(x2, axis=1, keepdims=True)
            pos = jnp.min(jnp.where(x2 == m, slot, nc), axis=1, keepdims=True)
            sel = slot == pos
            fv.append(m)
            fi.append(jnp.sum(jnp.where(sel, ci, 0), axis=1, keepdims=True))
            x2 = jnp.where(sel, -jnp.inf, x2)
        topv = jnp.concatenate(fv, axis=1)
        idx_ref[...] = jnp.concatenate(fi, axis=1)

        w = jnp.exp(topv - topv[:, :1])
        w = w / jnp.sum(w, axis=1, keepdims=True)

        amps = jax.lax.dot_general(
            hs, aw_ref[...], (((1,), (1,)), ((), ())),
            preferred_element_type=jnp.float32) + ab_ref[...]

        r8 = jax.lax.broadcasted_iota(jnp.int32, (K, AK), 0)
        c32 = jax.lax.broadcasted_iota(jnp.int32, (K, AK), 1)
        expand = (c32 // AMP_DIM == r8).astype(jnp.float32)
        w32 = jax.lax.dot_general(
            w, expand, (((1,), (0,)), ((), ())),
            preferred_element_type=jnp.float32)
        wa = amps * w32
        g = wa * wa
        p = jax.lax.broadcasted_iota(jnp.int32, (AK, AK), 0)
        q = jax.lax.broadcasted_iota(jnp.int32, (AK, AK), 1)
        gsum = (p // AMP_DIM == q // AMP_DIM).astype(jnp.float32)
        n2 = jax.lax.dot_general(
            g, gsum, (((1,), (0,)), ((), ())),
            preferred_element_type=jnp.float32)
        amp_ref[...] = wa / jnp.maximum(jnp.sqrt(n2), 1e-12)


@functools.partial(jax.jit, static_argnames=())
def kernel(hidden_states, score_w, score_b, amp_w, amp_b):
    batch, seq, d = hidden_states.shape
    rows = batch * seq
    hs2 = hidden_states.reshape(rows, d)
    sb2 = score_b.reshape(1, NUM_PRIMES)
    ab2 = amp_b.reshape(1, AK)
    nr = rows // ROW_BLOCK

    idx_out, amp_out = pl.pallas_call(
        _fused,
        grid=(nr, NUM_TILES),
        in_specs=[
            pl.BlockSpec((ROW_BLOCK, d), lambda i, j: (i, 0)),
            pl.BlockSpec((PRIME_TILE, d), lambda i, j: (j, 0)),
            pl.BlockSpec((1, PRIME_TILE), lambda i, j: (0, j)),
            pl.BlockSpec((AK, d), lambda i, j: (0, 0)),
            pl.BlockSpec((1, AK), lambda i, j: (0, 0)),
        ],
        out_specs=[
            pl.BlockSpec((ROW_BLOCK, K), lambda i, j: (i, 0)),
            pl.BlockSpec((ROW_BLOCK, AK), lambda i, j: (i, 0)),
        ],
        out_shape=[
            jax.ShapeDtypeStruct((rows, K), jnp.int32),
            jax.ShapeDtypeStruct((rows, AK), jnp.float32),
        ],
        scratch_shapes=[
            pltpu.VMEM((ROW_BLOCK, NUM_TILES * SLOT), jnp.float32),
            pltpu.VMEM((ROW_BLOCK, NUM_TILES * SLOT), jnp.int32),
        ],
        compiler_params=pltpu.CompilerParams(
            dimension_semantics=("parallel", "arbitrary")),
    )(hs2, score_w, sb2, amp_w, ab2)

    topk_indices = idx_out.reshape(batch, seq, K)
    amps = amp_out.reshape(batch, seq, K, AMP_DIM)
    return (topk_indices, amps)
